# baseline (speedup 1.0000x reference)
import functools

import jax
import jax.numpy as jnp
from jax import lax
from jax.experimental import pallas as pl
from jax.experimental.pallas import tpu as pltpu
from jax.experimental.pallas import tpu_sc as plsc

N_NODES = 10000
N_EDGES = 320000
D_FEAT = 128
N_OUT = 7

NPAD = 10240
KF = 8
KH = 4
NW = 32
EPW = N_EDGES // NW


def _tc_project(x_pad, w_t):
    blk = 2048

    def body(x_ref, w_ref, out_ref):
        out_ref[...] = lax.dot_general(
            w_ref[...], x_ref[...],
            (((1,), (1,)), ((), ())),
            preferred_element_type=jnp.float32,
        )

    return pl.pallas_call(
        body,
        grid=(NPAD // blk,),
        in_specs=[
            pl.BlockSpec((blk, D_FEAT), lambda i: (i, 0)),
            pl.BlockSpec((KF, D_FEAT), lambda i: (0, 0)),
        ],
        out_specs=pl.BlockSpec((KF, blk), lambda i: (0, i)),
        out_shape=jax.ShapeDtypeStruct((KF, NPAD), jnp.float32),
    )(x_pad, w_t)


def _sc_scatter(xw_t, src, dst, wgt):
    mesh = plsc.VectorSubcoreMesh(core_axis_name="c", subcore_axis_name="s")

    half = KH * NPAD

    @functools.partial(
        pl.kernel,
        mesh=mesh,
        out_type=jax.ShapeDtypeStruct((NW, KF * NPAD), jnp.float32),
        scratch_types=[
            pltpu.VMEM((half,), jnp.float32),
            pltpu.VMEM((half,), jnp.float32),
            pltpu.VMEM((EPW,), jnp.int32),
            pltpu.VMEM((EPW,), jnp.int32),
            pltpu.VMEM((EPW,), jnp.float32),
        ],
        compiler_params=pltpu.CompilerParams(needs_layout_passes=False),
    )
    def sc_kernel(xwt_hbm, src_hbm, dst_hbm, wgt_hbm, out_hbm,
                  table_v, accum_v, src_v, dst_v, wgt_v):
        wid = lax.axis_index("c") * 16 + lax.axis_index("s")
        pltpu.sync_copy(src_hbm.at[wid], src_v)
        pltpu.sync_copy(dst_hbm.at[wid], dst_v)
        pltpu.sync_copy(wgt_hbm.at[wid], wgt_v)

        for p in range(KF // KH):
            pltpu.sync_copy(xwt_hbm.at[pl.ds(p * half, half)], table_v)

            def zero_body(i, carry):
                accum_v[pl.ds(i * 16, 16)] = jnp.zeros((16,), jnp.float32)
                return carry

            lax.fori_loop(0, half // 16, zero_body, 0, unroll=8)

            def edge_body(j, carry):
                for g in range(5):
                    b = j * 80 + g * 16
                    sv = src_v[pl.ds(b, 16)]
                    dv = dst_v[pl.ds(b, 16)]
                    wv = wgt_v[pl.ds(b, 16)]
                    for k in range(KH):
                        vals = plsc.load_gather(table_v, [sv + (k * NPAD)])
                        plsc.addupdate_scatter(accum_v, [dv + (k * NPAD)], vals * wv)
                return carry

            lax.fori_loop(0, EPW // 80, edge_body, 0)

            pltpu.sync_copy(accum_v, out_hbm.at[wid, pl.ds(p * half, half)])

    return sc_kernel(xw_t.reshape(KF * NPAD), src, dst, wgt).reshape(NW, KF, NPAD)


def _tc_reduce_softmax(partials):
    blk = 1024

    def body(p_ref, out_ref):
        s = jnp.sum(p_ref[...], axis=0)
        valid = lax.broadcasted_iota(jnp.int32, (KF, blk), 0) < N_OUT
        m = jnp.max(jnp.where(valid, s, -jnp.inf), axis=0, keepdims=True)
        e = jnp.where(valid, jnp.exp(s - m), 0.0)
        out_ref[...] = e / jnp.sum(e, axis=0, keepdims=True)

    return pl.pallas_call(
        body,
        grid=(NPAD // blk,),
        in_specs=[pl.BlockSpec((NW, KF, blk), lambda i: (0, 0, i))],
        out_specs=pl.BlockSpec((KF, blk), lambda i: (0, i)),
        out_shape=jax.ShapeDtypeStruct((KF, NPAD), jnp.float32),
    )(partials)


def kernel(x, edge_index, edge_weight, W):
    x_pad = jnp.zeros((NPAD, D_FEAT), jnp.float32).at[:N_NODES].set(x)
    w_t = jnp.zeros((KF, D_FEAT), jnp.float32).at[:N_OUT].set(W.T)
    src = edge_index[0].astype(jnp.int32).reshape(NW, EPW)
    dst = edge_index[1].astype(jnp.int32).reshape(NW, EPW)
    wgt = edge_weight.reshape(NW, EPW)

    xw_t = _tc_project(x_pad, w_t)
    partials = _sc_scatter(xw_t, src, dst, wgt)
    sm = _tc_reduce_softmax(partials)
    return sm[:N_OUT, :N_NODES].T


# speedup vs baseline: 1.1790x; 1.1790x over previous
import functools

import jax
import jax.numpy as jnp
from jax import lax
from jax.experimental import pallas as pl
from jax.experimental.pallas import tpu as pltpu
from jax.experimental.pallas import tpu_sc as plsc

N_NODES = 10000
N_EDGES = 320000
D_FEAT = 128
N_OUT = 7

NPAD = 10240
KF = 8
KH = 4
NW = 32
EPW = N_EDGES // NW


def _tc_project(x, w_t):
    blk = 2048

    def body(x_ref, w_ref, out_ref):
        out_ref[...] = lax.dot_general(
            w_ref[...], x_ref[...],
            (((1,), (1,)), ((), ())),
            preferred_element_type=jnp.float32,
        )

    return pl.pallas_call(
        body,
        grid=(NPAD // blk,),
        in_specs=[
            pl.BlockSpec((blk, D_FEAT), lambda i: (i, 0)),
            pl.BlockSpec((KF, D_FEAT), lambda i: (0, 0)),
        ],
        out_specs=pl.BlockSpec((KF, blk), lambda i: (0, i)),
        out_shape=jax.ShapeDtypeStruct((KF, NPAD), jnp.float32),
    )(x, w_t)


def _sc_scatter(xw_t, edge_index, edge_weight):
    mesh = plsc.VectorSubcoreMesh(core_axis_name="c", subcore_axis_name="s")
    half = KH * NPAD

    @functools.partial(
        pl.kernel,
        mesh=mesh,
        out_type=jax.ShapeDtypeStruct((NW, KF * NPAD), jnp.float32),
        scratch_types=[
            pltpu.VMEM((half,), jnp.float32),
            pltpu.VMEM((half,), jnp.float32),
            pltpu.VMEM((EPW,), jnp.int32),
            pltpu.VMEM((EPW,), jnp.int32),
            pltpu.VMEM((EPW,), jnp.float32),
            pltpu.SemaphoreType.DMA,
        ],
        compiler_params=pltpu.CompilerParams(needs_layout_passes=False),
    )
    def sc_kernel(xwt_hbm, ei_hbm, ew_hbm, out_hbm,
                  table_v, accum_v, src_v, dst_v, wgt_v, sem):
        wid = lax.axis_index("c") * 16 + lax.axis_index("s")
        e0 = wid * EPW
        cps = [
            pltpu.async_copy(ei_hbm.at[pl.ds(e0, EPW)], src_v, sem),
            pltpu.async_copy(ei_hbm.at[pl.ds(N_EDGES + e0, EPW)], dst_v, sem),
            pltpu.async_copy(ew_hbm.at[pl.ds(e0, EPW)], wgt_v, sem),
            pltpu.async_copy(xwt_hbm.at[pl.ds(0, half)], table_v, sem),
        ]

        def zero_body(i, carry):
            accum_v[pl.ds(i * 16, 16)] = jnp.zeros((16,), jnp.float32)
            return carry

        def edge_body(j, carry):
            for g in range(5):
                b = j * 80 + g * 16
                sv = src_v[pl.ds(b, 16)]
                dv = dst_v[pl.ds(b, 16)]
                wv = wgt_v[pl.ds(b, 16)]
                for k in range(KH):
                    vals = plsc.load_gather(table_v, [sv + (k * NPAD)])
                    plsc.addupdate_scatter(accum_v, [dv + (k * NPAD)], vals * wv)
            return carry

        lax.fori_loop(0, half // 16, zero_body, 0, unroll=8)
        for cp in cps:
            cp.wait()
        lax.fori_loop(0, EPW // 80, edge_body, 0)

        cp_t = pltpu.async_copy(xwt_hbm.at[pl.ds(half, half)], table_v, sem)
        pltpu.sync_copy(accum_v, out_hbm.at[wid, pl.ds(0, half)])
        lax.fori_loop(0, half // 16, zero_body, 0, unroll=8)
        cp_t.wait()
        lax.fori_loop(0, EPW // 80, edge_body, 0)
        pltpu.sync_copy(accum_v, out_hbm.at[wid, pl.ds(half, half)])

    return sc_kernel(xw_t.reshape(KF * NPAD), edge_index.reshape(2 * N_EDGES),
                     edge_weight)


def _tc_reduce_softmax(partials):
    blk = 1024

    def body(p_ref, out_ref):
        s = jnp.sum(p_ref[...], axis=0)
        valid = lax.broadcasted_iota(jnp.int32, (KF, blk), 0) < N_OUT
        m = jnp.max(jnp.where(valid, s, -jnp.inf), axis=0, keepdims=True)
        e = jnp.where(valid, jnp.exp(s - m), 0.0)
        sm = e / jnp.sum(e, axis=0, keepdims=True)
        out_ref[...] = sm.T[:, :N_OUT]

    return pl.pallas_call(
        body,
        grid=(NPAD // blk,),
        in_specs=[pl.BlockSpec((NW, KF, blk), lambda i: (0, 0, i))],
        out_specs=pl.BlockSpec((blk, N_OUT), lambda i: (i, 0)),
        out_shape=jax.ShapeDtypeStruct((N_NODES, N_OUT), jnp.float32),
    )(partials)


def kernel(x, edge_index, edge_weight, W):
    w_t = jnp.zeros((KF, D_FEAT), jnp.float32).at[:N_OUT].set(W.T)
    ei = edge_index.astype(jnp.int32)
    xw_t = _tc_project(x, w_t)
    partials = _sc_scatter(xw_t, ei, edge_weight)
    return _tc_reduce_softmax(partials.reshape(NW, KF, NPAD))


# speedup vs baseline: 1.3257x; 1.1245x over previous
import functools

import jax
import jax.numpy as jnp
from jax import lax
from jax.experimental import pallas as pl
from jax.experimental.pallas import tpu as pltpu
from jax.experimental.pallas import tpu_sc as plsc

N_NODES = 10000
N_EDGES = 320000
D_FEAT = 128
N_OUT = 7

NPAD = 10240
KF = 8
KH = 4
NW = 32
EPW = N_EDGES // NW


def _tc_project(x, w_t):
    blk = 2048

    def body(x_ref, w_ref, out_ref):
        out_ref[...] = lax.dot_general(
            w_ref[...], x_ref[...],
            (((1,), (1,)), ((), ())),
            preferred_element_type=jnp.float32,
        )

    return pl.pallas_call(
        body,
        grid=(NPAD // blk,),
        in_specs=[
            pl.BlockSpec((blk, D_FEAT), lambda i: (i, 0)),
            pl.BlockSpec((KF, D_FEAT), lambda i: (0, 0)),
        ],
        out_specs=pl.BlockSpec((KF, blk), lambda i: (0, i)),
        out_shape=jax.ShapeDtypeStruct((KF, NPAD), jnp.float32),
    )(x, w_t)


def _sc_scatter(xw_t, edge_index, edge_weight):
    mesh = plsc.VectorSubcoreMesh(core_axis_name="c", subcore_axis_name="s")
    half = KH * NPAD

    @functools.partial(
        pl.kernel,
        mesh=mesh,
        out_type=jax.ShapeDtypeStruct((NW, KF * NPAD), jnp.float32),
        scratch_types=[
            pltpu.VMEM((half,), jnp.float32),
            pltpu.VMEM((half,), jnp.float32),
            pltpu.VMEM((EPW,), jnp.int32),
            pltpu.VMEM((EPW,), jnp.int32),
            pltpu.VMEM((EPW,), jnp.float32),
            pltpu.SemaphoreType.DMA,
        ],
        compiler_params=pltpu.CompilerParams(needs_layout_passes=False),
    )
    def sc_kernel(xwt_hbm, ei_hbm, ew_hbm, out_hbm,
                  table_v, accum_v, src_v, dst_v, wgt_v, sem):
        wid = lax.axis_index("c") * 16 + lax.axis_index("s")
        e0 = wid * EPW
        cps = [
            pltpu.async_copy(ei_hbm.at[pl.ds(e0, EPW)], src_v, sem),
            pltpu.async_copy(ei_hbm.at[pl.ds(N_EDGES + e0, EPW)], dst_v, sem),
            pltpu.async_copy(ew_hbm.at[pl.ds(e0, EPW)], wgt_v, sem),
            pltpu.async_copy(xwt_hbm.at[pl.ds(0, half)], table_v, sem),
        ]

        def zero_body(i, carry):
            accum_v[pl.ds(i * 16, 16)] = jnp.zeros((16,), jnp.float32)
            return carry

        def edge_body(j, carry):
            for g in range(5):
                b = j * 80 + g * 16
                sv = src_v[pl.ds(b, 16)]
                dv = dst_v[pl.ds(b, 16)]
                wv = wgt_v[pl.ds(b, 16)]
                for k in range(KH):
                    vals = plsc.load_gather(table_v, [sv + (k * NPAD)])
                    plsc.addupdate_scatter(accum_v, [dv + (k * NPAD)], vals * wv)
            return carry

        lax.fori_loop(0, half // 16, zero_body, 0, unroll=8)
        for cp in cps:
            cp.wait()
        lax.fori_loop(0, EPW // 80, edge_body, 0)

        cp_t = pltpu.async_copy(xwt_hbm.at[pl.ds(half, half)], table_v, sem)
        pltpu.sync_copy(accum_v, out_hbm.at[wid, pl.ds(0, half)])
        lax.fori_loop(0, half // 16, zero_body, 0, unroll=8)
        cp_t.wait()
        lax.fori_loop(0, EPW // 80, edge_body, 0)
        pltpu.sync_copy(accum_v, out_hbm.at[wid, pl.ds(half, half)])

    return sc_kernel(xw_t.reshape(KF * NPAD), edge_index.reshape(2 * N_EDGES),
                     edge_weight)


def _tc_reduce_softmax(partials):
    blk = 1024

    nblk = NPAD // blk

    def body(*refs):
        out_ref = refs[KF]
        s = jnp.concatenate([jnp.sum(refs[k][...], axis=0, keepdims=True)
                             for k in range(KF)], axis=0)
        valid = lax.broadcasted_iota(jnp.int32, (KF, blk), 0) < N_OUT
        m = jnp.max(jnp.where(valid, s, -jnp.inf), axis=0, keepdims=True)
        e = jnp.where(valid, jnp.exp(s - m), 0.0)
        sm = e / jnp.sum(e, axis=0, keepdims=True)
        out_ref[...] = sm.T[:, :N_OUT]

    return pl.pallas_call(
        body,
        grid=(nblk,),
        in_specs=[
            pl.BlockSpec((NW, blk), lambda i, k=k: (0, k * nblk + i))
            for k in range(KF)
        ],
        out_specs=pl.BlockSpec((blk, N_OUT), lambda i: (i, 0)),
        out_shape=jax.ShapeDtypeStruct((N_NODES, N_OUT), jnp.float32),
    )(*([partials] * KF))


def kernel(x, edge_index, edge_weight, W):
    w_t = jnp.zeros((KF, D_FEAT), jnp.float32).at[:N_OUT].set(W.T)
    ei = edge_index.astype(jnp.int32)
    xw_t = _tc_project(x, w_t)
    partials = _sc_scatter(xw_t, ei, edge_weight)
    return _tc_reduce_softmax(partials)


# speedup vs baseline: 1.7749x; 1.3389x over previous
import functools

import jax
import jax.numpy as jnp
from jax import lax
from jax.experimental import pallas as pl
from jax.experimental.pallas import tpu as pltpu
from jax.experimental.pallas import tpu_sc as plsc

N_NODES = 10000
N_EDGES = 320000
D_FEAT = 128
N_OUT = 7

NPAD = 10240
KF = 8
KH = 4
NW = 32
EPW = N_EDGES // NW


def _tc_project(x, w_t):
    blk = 2048

    def body(x_ref, w_ref, out_ref):
        out_ref[...] = lax.dot_general(
            w_ref[...], x_ref[...],
            (((1,), (1,)), ((), ())),
            preferred_element_type=jnp.float32,
        )

    return pl.pallas_call(
        body,
        grid=(NPAD // blk,),
        in_specs=[
            pl.BlockSpec((blk, D_FEAT), lambda i: (i, 0)),
            pl.BlockSpec((KF, D_FEAT), lambda i: (0, 0)),
        ],
        out_specs=pl.BlockSpec((KF, blk), lambda i: (0, i)),
        out_shape=jax.ShapeDtypeStruct((KF, NPAD), jnp.float32),
    )(x, w_t)


def _sc_scatter(xw_t, edge_index, edge_weight):
    mesh = plsc.VectorSubcoreMesh(core_axis_name="c", subcore_axis_name="s")
    half = KH * NPAD

    @functools.partial(
        pl.kernel,
        mesh=mesh,
        out_type=jax.ShapeDtypeStruct((NW, KF * NPAD), jnp.float32),
        scratch_types=[
            pltpu.VMEM((half,), jnp.float32),
            pltpu.VMEM((half,), jnp.float32),
            pltpu.VMEM((EPW,), jnp.int32),
            pltpu.VMEM((EPW,), jnp.int32),
            pltpu.VMEM((EPW,), jnp.float32),
            pltpu.SemaphoreType.DMA,
        ],
        compiler_params=pltpu.CompilerParams(needs_layout_passes=False),
    )
    def sc_kernel(xwt_hbm, ei_hbm, ew_hbm, out_hbm,
                  table_v, accum_v, src_v, dst_v, wgt_v, sem):
        wid = lax.axis_index("c") * 16 + lax.axis_index("s")
        e0 = wid * EPW
        cps = [
            pltpu.async_copy(ei_hbm.at[pl.ds(e0, EPW)], src_v, sem),
            pltpu.async_copy(ei_hbm.at[pl.ds(N_EDGES + e0, EPW)], dst_v, sem),
            pltpu.async_copy(ew_hbm.at[pl.ds(e0, EPW)], wgt_v, sem),
            pltpu.async_copy(xwt_hbm.at[pl.ds(0, half)], table_v, sem),
        ]

        def zero_accum():
            @plsc.parallel_loop(0, half, 16, unroll=8)
            def _(b):
                accum_v[pl.ds(b, 16)] = jnp.zeros((16,), jnp.float32)

        def edge_loop():
            @plsc.parallel_loop(0, EPW, 16, unroll=4)
            def _(b):
                sv = src_v[pl.ds(b, 16)]
                dv = dst_v[pl.ds(b, 16)]
                wv = wgt_v[pl.ds(b, 16)]
                for k in range(KH):
                    vals = plsc.load_gather(table_v, [sv + (k * NPAD)])
                    plsc.addupdate_scatter(accum_v, [dv + (k * NPAD)], vals * wv)

        zero_accum()
        for cp in cps:
            cp.wait()
        edge_loop()

        cp_t = pltpu.async_copy(xwt_hbm.at[pl.ds(half, half)], table_v, sem)
        pltpu.sync_copy(accum_v, out_hbm.at[wid, pl.ds(0, half)])
        zero_accum()
        cp_t.wait()
        edge_loop()
        pltpu.sync_copy(accum_v, out_hbm.at[wid, pl.ds(half, half)])

    return sc_kernel(xw_t.reshape(KF * NPAD), edge_index.reshape(2 * N_EDGES),
                     edge_weight)


def _tc_reduce_softmax(partials):
    blk = 1024

    nblk = NPAD // blk

    def body(*refs):
        out_ref = refs[KF]
        s = jnp.concatenate([jnp.sum(refs[k][...], axis=0, keepdims=True)
                             for k in range(KF)], axis=0)
        valid = lax.broadcasted_iota(jnp.int32, (KF, blk), 0) < N_OUT
        m = jnp.max(jnp.where(valid, s, -jnp.inf), axis=0, keepdims=True)
        e = jnp.where(valid, jnp.exp(s - m), 0.0)
        sm = e / jnp.sum(e, axis=0, keepdims=True)
        out_ref[...] = sm.T[:, :N_OUT]

    return pl.pallas_call(
        body,
        grid=(nblk,),
        in_specs=[
            pl.BlockSpec((NW, blk), lambda i, k=k: (0, k * nblk + i))
            for k in range(KF)
        ],
        out_specs=pl.BlockSpec((blk, N_OUT), lambda i: (i, 0)),
        out_shape=jax.ShapeDtypeStruct((N_NODES, N_OUT), jnp.float32),
    )(*([partials] * KF))


def kernel(x, edge_index, edge_weight, W):
    w_t = jnp.zeros((KF, D_FEAT), jnp.float32).at[:N_OUT].set(W.T)
    ei = edge_index.astype(jnp.int32)
    xw_t = _tc_project(x, w_t)
    partials = _sc_scatter(xw_t, ei, edge_weight)
    return _tc_reduce_softmax(partials)


# speedup vs baseline: 1.8096x; 1.0195x over previous
import functools

import jax
import jax.numpy as jnp
from jax import lax
from jax.experimental import pallas as pl
from jax.experimental.pallas import tpu as pltpu
from jax.experimental.pallas import tpu_sc as plsc

N_NODES = 10000
N_EDGES = 320000
D_FEAT = 128
N_OUT = 7

NPAD = 10240
KF = 8
KH = 4
NW = 32
EPW = N_EDGES // NW


def _tc_project(x, w_t, ei):
    blk = 2048
    eblk = N_EDGES // (NPAD // blk)

    def body(x_ref, w_ref, ei_ref, out_ref, src_ref, dst_ref):
        out_ref[...] = lax.dot_general(
            w_ref[...], x_ref[...],
            (((1,), (1,)), ((), ())),
            preferred_element_type=jnp.float32,
        )
        e = pl.program_id(0) * eblk
        src_ref[pl.ds(e, eblk)] = ei_ref[0]
        dst_ref[pl.ds(e, eblk)] = ei_ref[1]

    return pl.pallas_call(
        body,
        grid=(NPAD // blk,),
        in_specs=[
            pl.BlockSpec((blk, D_FEAT), lambda i: (i, 0)),
            pl.BlockSpec((KF, D_FEAT), lambda i: (0, 0)),
            pl.BlockSpec((2, eblk), lambda i: (0, i)),
        ],
        out_specs=[
            pl.BlockSpec((KF, blk), lambda i: (0, i)),
            pl.BlockSpec((N_EDGES,), lambda i: (0,)),
            pl.BlockSpec((N_EDGES,), lambda i: (0,)),
        ],
        out_shape=[
            jax.ShapeDtypeStruct((KF, NPAD), jnp.float32),
            jax.ShapeDtypeStruct((N_EDGES,), jnp.int32),
            jax.ShapeDtypeStruct((N_EDGES,), jnp.int32),
        ],
    )(x, w_t, ei)


def _sc_scatter(xw_t, src, dst, edge_weight):
    mesh = plsc.VectorSubcoreMesh(core_axis_name="c", subcore_axis_name="s")
    half = KH * NPAD

    @functools.partial(
        pl.kernel,
        mesh=mesh,
        out_type=jax.ShapeDtypeStruct((NW, KF * NPAD), jnp.float32),
        scratch_types=[
            pltpu.VMEM((half,), jnp.float32),
            pltpu.VMEM((half,), jnp.float32),
            pltpu.VMEM((EPW,), jnp.int32),
            pltpu.VMEM((EPW,), jnp.int32),
            pltpu.VMEM((EPW,), jnp.float32),
            pltpu.SemaphoreType.DMA,
        ],
        compiler_params=pltpu.CompilerParams(needs_layout_passes=False),
    )
    def sc_kernel(xwt_hbm, src_hbm, dst_hbm, ew_hbm, out_hbm,
                  table_v, accum_v, src_v, dst_v, wgt_v, sem):
        wid = lax.axis_index("c") * 16 + lax.axis_index("s")
        e0 = wid * EPW
        cps = [
            pltpu.async_copy(src_hbm.at[pl.ds(e0, EPW)], src_v, sem),
            pltpu.async_copy(dst_hbm.at[pl.ds(e0, EPW)], dst_v, sem),
            pltpu.async_copy(ew_hbm.at[pl.ds(e0, EPW)], wgt_v, sem),
            pltpu.async_copy(xwt_hbm.at[pl.ds(0, half)], table_v, sem),
        ]

        def zero_accum():
            @plsc.parallel_loop(0, half, 16, unroll=8)
            def _(b):
                accum_v[pl.ds(b, 16)] = jnp.zeros((16,), jnp.float32)

        def edge_loop():
            @plsc.parallel_loop(0, EPW, 16, unroll=4)
            def _(b):
                sv = src_v[pl.ds(b, 16)]
                dv = dst_v[pl.ds(b, 16)]
                wv = wgt_v[pl.ds(b, 16)]
                for k in range(KH):
                    vals = plsc.load_gather(table_v, [sv + (k * NPAD)])
                    plsc.addupdate_scatter(accum_v, [dv + (k * NPAD)], vals * wv)

        zero_accum()
        for cp in cps:
            cp.wait()
        edge_loop()

        cp_t = pltpu.async_copy(xwt_hbm.at[pl.ds(half, half)], table_v, sem)
        pltpu.sync_copy(accum_v, out_hbm.at[wid, pl.ds(0, half)])
        zero_accum()
        cp_t.wait()
        edge_loop()
        pltpu.sync_copy(accum_v, out_hbm.at[wid, pl.ds(half, half)])

    return sc_kernel(xw_t.reshape(KF * NPAD), src, dst, edge_weight)


def _tc_reduce_softmax(partials):
    blk = 1024

    nblk = NPAD // blk

    def body(*refs):
        out_ref = refs[KF]
        s = jnp.concatenate([jnp.sum(refs[k][...], axis=0, keepdims=True)
                             for k in range(KF)], axis=0)
        valid = lax.broadcasted_iota(jnp.int32, (KF, blk), 0) < N_OUT
        m = jnp.max(jnp.where(valid, s, -jnp.inf), axis=0, keepdims=True)
        e = jnp.where(valid, jnp.exp(s - m), 0.0)
        sm = e / jnp.sum(e, axis=0, keepdims=True)
        out_ref[...] = sm.T[:, :N_OUT]

    return pl.pallas_call(
        body,
        grid=(nblk,),
        in_specs=[
            pl.BlockSpec((NW, blk), lambda i, k=k: (0, k * nblk + i))
            for k in range(KF)
        ],
        out_specs=pl.BlockSpec((blk, N_OUT), lambda i: (i, 0)),
        out_shape=jax.ShapeDtypeStruct((N_NODES, N_OUT), jnp.float32),
    )(*([partials] * KF))


def kernel(x, edge_index, edge_weight, W):
    w_t = jnp.zeros((KF, D_FEAT), jnp.float32).at[:N_OUT].set(W.T)
    ei = edge_index.astype(jnp.int32)
    xw_t, src, dst = _tc_project(x, w_t, ei)
    partials = _sc_scatter(xw_t, src, dst, edge_weight)
    return _tc_reduce_softmax(partials)
